# baseline (device time: 317202 ns/iter reference)
import jax
import jax.numpy as jnp
from jax import lax
from jax.experimental import pallas as pl
from jax.experimental.pallas import tpu as pltpu

N_DEV = 32
R_HOPS = 13
L_HOPS = 13
CHORD_DS = (14, 15, 16, -14, -15)


def _ring_mesh(p):
    q = jnp.where(p < 16, p, 31 - p)
    x = jnp.where(p < 16, 0, 1)
    y = q // 4
    r = q % 4
    z = jnp.where(y % 2 == 0, r, 3 - r)
    return 8 * z + 2 * y + jnp.bitwise_xor(x, y % 2)


def _pos_of(m):
    z = m // 8
    pi = m % 8
    y = pi // 2
    x = jnp.bitwise_xor(pi % 2, y % 2)
    q = 4 * y + jnp.where(y % 2 == 0, z, 3 - z)
    return jnp.where(x == 0, q, 31 - q)


def kernel(x, w_mat):
    m_per, k = x.shape
    _, n_per = w_mat.shape
    m_full = N_DEV * m_per

    def body(x_ref, w_ref, out_ref, xfull, amax_buf,
             r_send_sems, r_recv_sems, l_send_sems, l_recv_sems,
             c_send_sems, c_recv_sems, amax_send_sems, amax_recv_sems):
        my = lax.axis_index("i")
        pos = _pos_of(my)
        right = _ring_mesh((pos + 1) % N_DEV)
        left = _ring_mesh((pos - 1) % N_DEV)

        bsem = pltpu.get_barrier_semaphore()
        for j in range(1, N_DEV):
            pl.semaphore_signal(
                bsem, inc=1,
                device_id=((my + j) % N_DEV,),
                device_id_type=pl.DeviceIdType.MESH,
            )
        pl.semaphore_wait(bsem, N_DEV - 1)

        my_sl = pl.ds(my * m_per, m_per)
        xfull[my_sl, :] = x_ref[...].astype(jnp.bfloat16)

        chord_rdmas = []
        for ci, d in enumerate(CHORD_DS):
            tgt = _ring_mesh((pos + d) % N_DEV)
            rdma = pltpu.make_async_remote_copy(
                src_ref=xfull.at[my_sl, :],
                dst_ref=xfull.at[my_sl, :],
                send_sem=c_send_sems.at[ci],
                recv_sem=c_recv_sems.at[ci],
                device_id=(tgt,),
                device_id_type=pl.DeviceIdType.MESH,
            )
            rdma.start()
            chord_rdmas.append(rdma)

        rdmas = []
        for h in range(R_HOPS):
            o_r = _ring_mesh((pos - h) % N_DEV)
            sl = pl.ds(o_r * m_per, m_per)
            r_rdma = pltpu.make_async_remote_copy(
                src_ref=xfull.at[sl, :],
                dst_ref=xfull.at[sl, :],
                send_sem=r_send_sems.at[h],
                recv_sem=r_recv_sems.at[h],
                device_id=(right,),
                device_id_type=pl.DeviceIdType.MESH,
            )
            r_rdma.start()
            rdmas.append(r_rdma)
            l_rdma = None
            if h < L_HOPS:
                o_l = _ring_mesh((pos + h) % N_DEV)
                sll = pl.ds(o_l * m_per, m_per)
                l_rdma = pltpu.make_async_remote_copy(
                    src_ref=xfull.at[sll, :],
                    dst_ref=xfull.at[sll, :],
                    send_sem=l_send_sems.at[h],
                    recv_sem=l_recv_sems.at[h],
                    device_id=(left,),
                    device_id_type=pl.DeviceIdType.MESH,
                )
                l_rdma.start()
                rdmas.append(l_rdma)
            r_rdma.wait_recv()
            if l_rdma is not None:
                l_rdma.wait_recv()
        for rdma in chord_rdmas:
            rdma.wait_recv()
        for rdma in rdmas + chord_rdmas:
            rdma.wait_send()

        y = jnp.dot(xfull[...], w_ref[...].astype(jnp.bfloat16),
                    preferred_element_type=jnp.float32)
        y = jnp.maximum(y, 0.0)
        out_ref[...] = y

        running = jnp.max(y)
        amax_buf[pl.ds(my, 1), :, :] = jnp.full((1, 8, 128), running,
                                                jnp.float32)
        amax_rdmas = []
        for j in range(1, N_DEV):
            rdma = pltpu.make_async_remote_copy(
                src_ref=amax_buf.at[pl.ds(my, 1)],
                dst_ref=amax_buf.at[pl.ds(my, 1)],
                send_sem=amax_send_sems.at[j - 1],
                recv_sem=amax_recv_sems.at[j - 1],
                device_id=((my + j) % N_DEV,),
                device_id_type=pl.DeviceIdType.MESH,
            )
            rdma.start()
            amax_rdmas.append(rdma)
        for rdma in amax_rdmas:
            rdma.wait_recv()
        for rdma in amax_rdmas:
            rdma.wait_send()
        gmax = jnp.max(amax_buf[:, 0, 0])

        scale = gmax / 448.0
        q = jnp.minimum(out_ref[...] / scale, 448.0)
        q = q.astype(jnp.float8_e4m3fn)
        out_ref[...] = q.astype(jnp.float32) * scale

    return pl.pallas_call(
        body,
        out_shape=jax.ShapeDtypeStruct((m_full, n_per), jnp.float32),
        in_specs=[
            pl.BlockSpec(memory_space=pltpu.VMEM),
            pl.BlockSpec(memory_space=pltpu.VMEM),
        ],
        out_specs=pl.BlockSpec(memory_space=pltpu.VMEM),
        scratch_shapes=[
            pltpu.VMEM((m_full, k), jnp.bfloat16),
            pltpu.VMEM((N_DEV, 8, 128), jnp.float32),
            pltpu.SemaphoreType.DMA((R_HOPS,)),
            pltpu.SemaphoreType.DMA((R_HOPS,)),
            pltpu.SemaphoreType.DMA((L_HOPS,)),
            pltpu.SemaphoreType.DMA((L_HOPS,)),
            pltpu.SemaphoreType.DMA((len(CHORD_DS),)),
            pltpu.SemaphoreType.DMA((len(CHORD_DS),)),
            pltpu.SemaphoreType.DMA((N_DEV - 1,)),
            pltpu.SemaphoreType.DMA((N_DEV - 1,)),
        ],
        compiler_params=pltpu.CompilerParams(
            collective_id=0, vmem_limit_bytes=96 * 1024 * 1024,
        ),
    )(x, w_mat)


# device time: 226664 ns/iter; 1.3994x vs baseline; 1.3994x over previous
import jax
import jax.numpy as jnp
from jax import lax
from jax.experimental import pallas as pl
from jax.experimental.pallas import tpu as pltpu

N_DEV = 32
HOPS = 16


def _ring_mesh(p):
    q = jnp.where(p < 16, p, 31 - p)
    x = jnp.where(p < 16, 0, 1)
    y = q // 4
    r = q % 4
    z = jnp.where(y % 2 == 0, r, 3 - r)
    return 8 * z + 2 * y + jnp.bitwise_xor(x, y % 2)


def _pos_of(m):
    z = m // 8
    pi = m % 8
    y = pi // 2
    x = jnp.bitwise_xor(pi % 2, y % 2)
    q = 4 * y + jnp.where(y % 2 == 0, z, 3 - z)
    return jnp.where(x == 0, q, 31 - q)


def kernel(x, w_mat):
    m_per, k = x.shape
    _, n_per = w_mat.shape
    m_full = N_DEV * m_per
    half = m_per // 2

    def body(x_ref, w_ref, out_ref, xfull, amax_buf,
             r_send_sems, r_recv_sems, l_send_sems, l_recv_sems,
             amax_send_sems, amax_recv_sems):
        my = lax.axis_index("i")
        pos = _pos_of(my)
        right = _ring_mesh((pos + 1) % N_DEV)
        left = _ring_mesh((pos - 1) % N_DEV)
        w = w_ref[...].astype(jnp.bfloat16)

        bsem = pltpu.get_barrier_semaphore()
        for j in range(1, N_DEV):
            pl.semaphore_signal(
                bsem, inc=1,
                device_id=((my + j) % N_DEV,),
                device_id_type=pl.DeviceIdType.MESH,
            )
        pl.semaphore_wait(bsem, N_DEV - 1)

        xfull[pl.ds(my * m_per, m_per), :] = x_ref[...].astype(jnp.bfloat16)

        def dot_block(origin, running):
            rows = pl.ds(origin * m_per, m_per)
            yb = jnp.dot(xfull[rows, :], w,
                         preferred_element_type=jnp.float32)
            yb = jnp.maximum(yb, 0.0)
            out_ref[rows, :] = yb
            return jnp.maximum(running, jnp.max(yb))

        running = jnp.float32(0.0)
        rdmas = []
        for h in range(HOPS):
            o_r = _ring_mesh((pos - h) % N_DEV)
            o_l = _ring_mesh((pos + h) % N_DEV)
            if h < HOPS - 1:
                sl_r = pl.ds(o_r * m_per, m_per)
                sl_l = pl.ds(o_l * m_per, m_per)
            else:
                sl_r = pl.ds(o_r * m_per, half)
                sl_l = pl.ds(o_l * m_per + half, half)
            r_rdma = pltpu.make_async_remote_copy(
                src_ref=xfull.at[sl_r, :],
                dst_ref=xfull.at[sl_r, :],
                send_sem=r_send_sems.at[h],
                recv_sem=r_recv_sems.at[h],
                device_id=(right,),
                device_id_type=pl.DeviceIdType.MESH,
            )
            r_rdma.start()
            l_rdma = pltpu.make_async_remote_copy(
                src_ref=xfull.at[sl_l, :],
                dst_ref=xfull.at[sl_l, :],
                send_sem=l_send_sems.at[h],
                recv_sem=l_recv_sems.at[h],
                device_id=(left,),
                device_id_type=pl.DeviceIdType.MESH,
            )
            l_rdma.start()
            rdmas.extend([r_rdma, l_rdma])
            if h == 0:
                running = dot_block(my, running)
            else:
                running = dot_block(o_r, running)
                running = dot_block(o_l, running)
            r_rdma.wait_recv()
            l_rdma.wait_recv()
        running = dot_block(_ring_mesh((pos + 16) % N_DEV), running)

        amax_buf[pl.ds(my, 1), :, :] = jnp.full((1, 8, 128), running,
                                                jnp.float32)
        amax_rdmas = []
        for j in range(1, N_DEV):
            rdma = pltpu.make_async_remote_copy(
                src_ref=amax_buf.at[pl.ds(my, 1)],
                dst_ref=amax_buf.at[pl.ds(my, 1)],
                send_sem=amax_send_sems.at[j - 1],
                recv_sem=amax_recv_sems.at[j - 1],
                device_id=((my + j) % N_DEV,),
                device_id_type=pl.DeviceIdType.MESH,
            )
            rdma.start()
            amax_rdmas.append(rdma)
        for rdma in amax_rdmas:
            rdma.wait_recv()
        gmax = jnp.max(amax_buf[:, 0, 0])

        scale = gmax / 448.0
        q = jnp.minimum(out_ref[...] / scale, 448.0)
        q = q.astype(jnp.float8_e4m3fn)
        out_ref[...] = q.astype(jnp.float32) * scale

        for rdma in rdmas + amax_rdmas:
            rdma.wait_send()

    return pl.pallas_call(
        body,
        out_shape=jax.ShapeDtypeStruct((m_full, n_per), jnp.float32),
        in_specs=[
            pl.BlockSpec(memory_space=pltpu.VMEM),
            pl.BlockSpec(memory_space=pltpu.VMEM),
        ],
        out_specs=pl.BlockSpec(memory_space=pltpu.VMEM),
        scratch_shapes=[
            pltpu.VMEM((m_full, k), jnp.bfloat16),
            pltpu.VMEM((N_DEV, 8, 128), jnp.float32),
            pltpu.SemaphoreType.DMA((HOPS,)),
            pltpu.SemaphoreType.DMA((HOPS,)),
            pltpu.SemaphoreType.DMA((HOPS,)),
            pltpu.SemaphoreType.DMA((HOPS,)),
            pltpu.SemaphoreType.DMA((N_DEV - 1,)),
            pltpu.SemaphoreType.DMA((N_DEV - 1,)),
        ],
        compiler_params=pltpu.CompilerParams(
            collective_id=0, vmem_limit_bytes=96 * 1024 * 1024,
        ),
    )(x, w_mat)
